# Initial kernel scaffold; baseline (speedup 1.0000x reference)
import jax
import jax.numpy as jnp
from jax.experimental import pallas as pl


def kernel(input_ids, embed_weight):
    raise NotImplementedError("write your pallas kernel here")


# speedup vs baseline: 9.0496x; 9.0496x over previous
import functools

import jax
import jax.numpy as jnp
from jax import lax
from jax.experimental import pallas as pl
from jax.experimental.pallas import tpu as pltpu
from jax.experimental.pallas import tpu_sc as plsc

VOCAB = 100000
EMBED_DIM = 64
BATCH = 4096
SEQ = 200

NUM_CORES = 2
NUM_SUBCORES = 16
NUM_WORKERS = NUM_CORES * NUM_SUBCORES
ROWS_PER_WORKER = BATCH // NUM_WORKERS
IDX_PER_WORKER = ROWS_PER_WORKER * SEQ
CHUNK_A = 128
CHUNK_B = SEQ - CHUNK_A

_LANES = 16
_DSLICES = EMBED_DIM // _LANES


def _sc_mean_pool(idx_flat, embed_weight):
    mesh = plsc.VectorSubcoreMesh(core_axis_name="c", subcore_axis_name="s")

    @functools.partial(
        pl.kernel,
        mesh=mesh,
        out_type=jax.ShapeDtypeStruct((BATCH, EMBED_DIM), jnp.float32),
        compiler_params=pltpu.CompilerParams(use_tc_tiling_on_sc=False),
        scratch_types=[
            pltpu.VMEM((IDX_PER_WORKER,), jnp.int32),
            pltpu.VMEM((SEQ, EMBED_DIM), jnp.float32),
            pltpu.VMEM((EMBED_DIM,), jnp.float32),
            pltpu.SemaphoreType.DMA,
            pltpu.SemaphoreType.DMA,
        ],
    )
    def k(table_hbm, idx_hbm, out_hbm, idx_v, rows_v, stage_v, sem_a, sem_b):
        wid = lax.axis_index("s") * NUM_CORES + lax.axis_index("c")
        base = wid * IDX_PER_WORKER
        pltpu.sync_copy(idx_hbm.at[pl.ds(base, IDX_PER_WORKER)], idx_v)

        @pl.loop(0, ROWS_PER_WORKER)
        def _(r):
            off = r * SEQ
            cp_a = pltpu.async_copy(
                table_hbm.at[idx_v.at[pl.ds(off, CHUNK_A)]],
                rows_v.at[pl.ds(0, CHUNK_A)],
                sem_a,
            )
            cp_b = pltpu.async_copy(
                table_hbm.at[idx_v.at[pl.ds(off + CHUNK_A, CHUNK_B)]],
                rows_v.at[pl.ds(CHUNK_A, CHUNK_B)],
                sem_b,
            )
            cp_a.wait()
            cp_b.wait()

            zeros = jnp.zeros((_LANES,), jnp.float32)

            def body(l, accs):
                new = []
                for d in range(_DSLICES):
                    new.append(accs[d] + rows_v[l, pl.ds(d * _LANES, _LANES)])
                return tuple(new)

            accs = lax.fori_loop(0, SEQ, body, (zeros,) * _DSLICES)
            scale = jnp.float32(1.0 / SEQ)
            for d in range(_DSLICES):
                stage_v[pl.ds(d * _LANES, _LANES)] = accs[d] * scale

            row = wid * ROWS_PER_WORKER + r
            pltpu.sync_copy(stage_v, out_hbm.at[row])

    return k(embed_weight, idx_flat)


def kernel(input_ids, embed_weight):
    idx_flat = input_ids.reshape(-1)
    return _sc_mean_pool(idx_flat, embed_weight)


# speedup vs baseline: 13.7123x; 1.5152x over previous
import functools

import jax
import jax.numpy as jnp
from jax import lax
from jax.experimental import pallas as pl
from jax.experimental.pallas import tpu as pltpu
from jax.experimental.pallas import tpu_sc as plsc

VOCAB = 100000
EMBED_DIM = 64
BATCH = 4096
SEQ = 200

NUM_CORES = 2
NUM_SUBCORES = 16
NUM_WORKERS = NUM_CORES * NUM_SUBCORES
ROWS_PER_WORKER = BATCH // NUM_WORKERS
IDX_PER_WORKER = ROWS_PER_WORKER * SEQ
CHUNK_A = 128
CHUNK_B = SEQ - CHUNK_A

_LANES = 16
_DSLICES = EMBED_DIM // _LANES


def _sc_mean_pool(idx_flat, embed_weight):
    mesh = plsc.VectorSubcoreMesh(core_axis_name="c", subcore_axis_name="s")

    @functools.partial(
        pl.kernel,
        mesh=mesh,
        out_type=jax.ShapeDtypeStruct((BATCH, EMBED_DIM), jnp.float32),
        compiler_params=pltpu.CompilerParams(use_tc_tiling_on_sc=False),
        scratch_types=[
            pltpu.VMEM((IDX_PER_WORKER,), jnp.int32),
            pltpu.VMEM((SEQ, EMBED_DIM), jnp.float32),
            pltpu.VMEM((SEQ, EMBED_DIM), jnp.float32),
            pltpu.VMEM((ROWS_PER_WORKER, EMBED_DIM), jnp.float32),
            pltpu.SemaphoreType.DMA,
            pltpu.SemaphoreType.DMA,
        ],
    )
    def k(table_hbm, idx_hbm, out_hbm, idx_v, buf0, buf1, stage_v, sem0, sem1):
        wid = lax.axis_index("s") * NUM_CORES + lax.axis_index("c")
        base = wid * IDX_PER_WORKER
        pltpu.sync_copy(idx_hbm.at[pl.ds(base, IDX_PER_WORKER)], idx_v)

        def issue(r, buf, sem):
            off = r * SEQ
            pltpu.async_copy(
                table_hbm.at[idx_v.at[pl.ds(off, CHUNK_A)]],
                buf.at[pl.ds(0, CHUNK_A)],
                sem,
            )
            pltpu.async_copy(
                table_hbm.at[idx_v.at[pl.ds(off + CHUNK_A, CHUNK_B)]],
                buf.at[pl.ds(CHUNK_A, CHUNK_B)],
                sem,
            )

        def wait(buf, sem):
            pltpu.make_async_copy(table_hbm.at[pl.ds(0, SEQ)], buf, sem).wait()

        def reduce_store(r, buf):
            zeros = jnp.zeros((_LANES,), jnp.float32)

            def body(l, accs):
                return tuple(
                    accs[d] + buf[l, pl.ds(d * _LANES, _LANES)]
                    for d in range(_DSLICES)
                )

            accs = lax.fori_loop(0, SEQ, body, (zeros,) * _DSLICES)
            scale = jnp.float32(1.0 / SEQ)
            for d in range(_DSLICES):
                stage_v[r, pl.ds(d * _LANES, _LANES)] = accs[d] * scale

        issue(0, buf0, sem0)

        @pl.loop(0, ROWS_PER_WORKER, step=2)
        def _(g):
            issue(g + 1, buf1, sem1)
            wait(buf0, sem0)
            reduce_store(g, buf0)

            @pl.when(g + 2 < ROWS_PER_WORKER)
            def _():
                issue(g + 2, buf0, sem0)

            wait(buf1, sem1)
            reduce_store(g + 1, buf1)

        pltpu.sync_copy(
            stage_v, out_hbm.at[pl.ds(wid * ROWS_PER_WORKER, ROWS_PER_WORKER)]
        )

    return k(embed_weight, idx_flat)


def kernel(input_ids, embed_weight):
    idx_flat = input_ids.reshape(-1)
    return _sc_mean_pool(idx_flat, embed_weight)
